# baseline (speedup 1.0000x reference)
import jax
import jax.numpy as jnp
from jax.experimental import pallas as pl
from jax.experimental.pallas import tpu as pltpu

WINDOW = 50
BATCH = 4096
DIM = 128
SLOTS = 2


def _body(ptr_ref, x_ref, data_ref, out_ref):
    i = pl.program_id(0)
    p = ptr_ref[0]
    out_ref[...] = data_ref[...]

    @pl.when(i == p // SLOTS)
    def _overwrite():
        out_ref[pl.ds((p % SLOTS) * BATCH, BATCH), :] = x_ref[...]


def kernel(x, data, pointer):
    ptr = jnp.atleast_1d(jnp.asarray(pointer, dtype=jnp.int32))
    flat = data.reshape(WINDOW * BATCH, DIM)
    grid_spec = pltpu.PrefetchScalarGridSpec(
        num_scalar_prefetch=1,
        grid=(WINDOW // SLOTS,),
        in_specs=[
            pl.BlockSpec((BATCH, DIM), lambda i, p: (0, 0)),
            pl.BlockSpec((SLOTS * BATCH, DIM), lambda i, p: (i, 0)),
        ],
        out_specs=pl.BlockSpec((SLOTS * BATCH, DIM), lambda i, p: (i, 0)),
    )
    out = pl.pallas_call(
        _body,
        grid_spec=grid_spec,
        out_shape=jax.ShapeDtypeStruct((WINDOW * BATCH, DIM), jnp.float32),
        compiler_params=pltpu.CompilerParams(
            dimension_semantics=("arbitrary",),
        ),
    )(ptr, x, flat)
    return out.reshape(WINDOW, BATCH, DIM)


# speedup vs baseline: 1.0328x; 1.0328x over previous
import jax
import jax.numpy as jnp
from jax.experimental import pallas as pl
from jax.experimental.pallas import tpu as pltpu

WINDOW = 50
BATCH = 4096
DIM = 128
SLOTS = 5


def _body(ptr_ref, x_ref, data_ref, out_ref):
    i = pl.program_id(0)
    p = ptr_ref[0]
    out_ref[...] = data_ref[...]

    @pl.when(i == p // SLOTS)
    def _overwrite():
        out_ref[pl.ds((p % SLOTS) * BATCH, BATCH), :] = x_ref[...]


def kernel(x, data, pointer):
    ptr = jnp.atleast_1d(jnp.asarray(pointer, dtype=jnp.int32))
    flat = data.reshape(WINDOW * BATCH, DIM)
    grid_spec = pltpu.PrefetchScalarGridSpec(
        num_scalar_prefetch=1,
        grid=(WINDOW // SLOTS,),
        in_specs=[
            pl.BlockSpec((BATCH, DIM), lambda i, p: (0, 0)),
            pl.BlockSpec((SLOTS * BATCH, DIM), lambda i, p: (i, 0)),
        ],
        out_specs=pl.BlockSpec((SLOTS * BATCH, DIM), lambda i, p: (i, 0)),
    )
    out = pl.pallas_call(
        _body,
        grid_spec=grid_spec,
        out_shape=jax.ShapeDtypeStruct((WINDOW * BATCH, DIM), jnp.float32),
        compiler_params=pltpu.CompilerParams(
            dimension_semantics=("arbitrary",),
        ),
    )(ptr, x, flat)
    return out.reshape(WINDOW, BATCH, DIM)
